# Initial kernel scaffold; baseline (speedup 1.0000x reference)
import jax
import jax.numpy as jnp
from jax.experimental import pallas as pl


def kernel(x, edge_index, W1, b1, W2, b2):
    raise NotImplementedError("write your pallas kernel here")


# speedup vs baseline: 12.0480x; 12.0480x over previous
import jax
import jax.numpy as jnp
from jax import lax
from jax.experimental import pallas as pl
from jax.experimental.pallas import tpu as pltpu
from jax.experimental.pallas import tpu_sc as plsc

N = 10000
F_IN = 128
E = 320000
NSC = 2
NTILE = 16
CHUNK = 632
NP = NTILE * CHUNK
BE = 80

_MESH = dict(core_axis_name="c", subcore_axis_name="s")
_SC_PARAMS = pltpu.CompilerParams(use_tc_tiling_on_sc=False)


def _sc_mesh():
    return plsc.VectorSubcoreMesh(**_MESH)


def _deg_call(cols):
    ept = E // (NSC * NTILE)
    nb = ept // BE

    def body(cols_h, pdeg_out, dacc, cols_all, colsr, onesr, ibuf, sem):
        c = lax.axis_index("c")
        sid = lax.axis_index("s")
        v0 = sid * CHUNK
        cf = (1 - c).astype(jnp.float32)
        for j in range(BE // 16):
            onesr[pl.ds(j * 16, 16)] = jnp.ones((16,), jnp.float32)
        for j in range(640 // 16):
            ibuf[pl.ds(j * 16, 16)] = jnp.ones((16,), jnp.float32) * cf
        pltpu.sync_copy(ibuf.at[pl.ds(0, CHUNK)], dacc.at[pl.ds(v0, CHUNK)])
        e0 = c * (E // 2) + sid * ept
        pltpu.sync_copy(cols_h.at[pl.ds(e0, ept)], cols_all)
        plsc.subcore_barrier()

        def blk(b, carry):
            base = b * BE
            for j in range(BE // 16):
                colsr[pl.ds(j * 16, 16)] = cols_all[pl.ds(base + j * 16, 16)]
            pltpu.sync_copy(onesr, dacc.at[colsr], add=True)
            return carry

        lax.fori_loop(0, nb, blk, 0)
        plsc.subcore_barrier()
        pltpu.sync_copy(dacc.at[pl.ds(v0, CHUNK)], ibuf.at[pl.ds(0, CHUNK)])
        pltpu.sync_copy(ibuf.at[pl.ds(0, CHUNK)],
                        pdeg_out.at[pl.ds(c * NP + v0, CHUNK)])

    kfn = pl.kernel(
        body,
        out_type=jax.ShapeDtypeStruct((NSC * NP,), jnp.float32),
        mesh=_sc_mesh(),
        compiler_params=_SC_PARAMS,
        scratch_types=[
            pltpu.VMEM_SHARED((NP,), jnp.float32),
            pltpu.VMEM((ept,), jnp.int32),
            pltpu.VMEM((BE,), jnp.int32),
            pltpu.VMEM((BE,), jnp.float32),
            pltpu.VMEM((640,), jnp.float32),
            pltpu.SemaphoreType.DMA,
        ],
    )
    return kfn(cols)


def _make_prop(w, with_scale):
    ept = E // NTILE
    nb = ept // BE

    def body(hflat, rows_h, cols_h, *rest):
        if with_scale:
            s_h, outf, acc, rows_all, cols_all, colsr, gidxr, gbuf, iobuf, sr, sem = rest
        else:
            outf, acc, rows_all, cols_all, colsr, gidxr, gbuf, iobuf, sr, sem = rest
        c = lax.axis_index("c")
        sid = lax.axis_index("s")
        v0 = sid * CHUNK
        pltpu.sync_copy(hflat.at[pl.ds(c * NP + v0, CHUNK)], iobuf)
        pltpu.sync_copy(iobuf, acc.at[pl.ds(v0, CHUNK)])
        if with_scale:
            pltpu.sync_copy(s_h.at[pl.ds(v0, CHUNK)], sr)
        e0 = sid * ept
        pltpu.sync_copy(rows_h.at[pl.ds(e0, ept)], rows_all)
        pltpu.sync_copy(cols_h.at[pl.ds(e0, ept)], cols_all)
        plsc.subcore_barrier()

        def blk(b, carry):
            base = b * BE
            for j in range(BE // 16):
                rv = rows_all[pl.ds(base + j * 16, 16)]
                gidxr[pl.ds(j * 16, 16)] = rv + c * NP
                colsr[pl.ds(j * 16, 16)] = cols_all[pl.ds(base + j * 16, 16)]
            pltpu.async_copy(hflat.at[gidxr], gbuf, sem).wait()
            pltpu.sync_copy(gbuf, acc.at[colsr], add=True)
            return carry

        lax.fori_loop(0, nb, blk, 0)
        plsc.subcore_barrier()
        pltpu.sync_copy(acc.at[pl.ds(v0, CHUNK)], iobuf)
        if with_scale:
            def grpf(g, carry):
                sv = sr[pl.ds(g * 16, 16)]
                for l in range(16):
                    s_l = sv[l]
                    for j in range(w // 16):
                        r = g * 16 + l
                        iobuf[r, pl.ds(j * 16, 16)] = iobuf[r, pl.ds(j * 16, 16)] * s_l
                return carry
            lax.fori_loop(0, CHUNK // 16, grpf, 0)

            sv = sr[pl.ds(CHUNK - 16, 16)]
            for l in range(8, 16):
                s_l = sv[l]
                for j in range(w // 16):
                    r = CHUNK - 16 + l
                    iobuf[r, pl.ds(j * 16, 16)] = iobuf[r, pl.ds(j * 16, 16)] * s_l
        pltpu.sync_copy(iobuf, outf.at[pl.ds(c * NP + v0, CHUNK)])

    kfn = pl.kernel(
        body,
        out_type=jax.ShapeDtypeStruct((NSC * NP, w), jnp.float32),
        mesh=_sc_mesh(),
        compiler_params=_SC_PARAMS,
        scratch_types=[
            pltpu.VMEM_SHARED((NP, w), jnp.float32),
            pltpu.VMEM((ept,), jnp.int32),
            pltpu.VMEM((ept,), jnp.int32),
            pltpu.VMEM((BE,), jnp.int32),
            pltpu.VMEM((BE,), jnp.int32),
            pltpu.VMEM((BE, w), jnp.float32),
            pltpu.VMEM((CHUNK, w), jnp.float32),
            pltpu.VMEM((CHUNK,), jnp.float32),
            pltpu.SemaphoreType.DMA,
        ],
    )
    return kfn


def _prop(h3, rows, cols, s, w):
    hflat = h3.reshape(NSC * NP, w)
    kfn = _make_prop(w, s is not None)
    if s is not None:
        out = kfn(hflat, rows, cols, s.reshape(NP))
    else:
        out = kfn(hflat, rows, cols)
    return out.reshape(NSC, NP, w)


def _tc_pre(pdeg_t, xp):
    def body(pd_ref, x_ref, x0_ref, dis_ref, inv_ref):
        pd = pd_ref[...]
        deg = pd[:, 0:1] + pd[:, 1:2] - 1.0
        dis = lax.rsqrt(deg)
        dis_ref[...] = dis
        inv_ref[...] = 1.0 / deg
        x0 = x_ref[...] * dis
        x0_ref[0] = x0[:, :64]
        x0_ref[1] = x0[:, 64:]

    return pl.pallas_call(
        body,
        grid=(NP // CHUNK,),
        in_specs=[
            pl.BlockSpec((CHUNK, NSC), lambda i: (i, 0)),
            pl.BlockSpec((CHUNK, F_IN), lambda i: (i, 0)),
        ],
        out_specs=[
            pl.BlockSpec((NSC, CHUNK, 64), lambda i: (0, i, 0)),
            pl.BlockSpec((CHUNK, 1), lambda i: (i, 0)),
            pl.BlockSpec((CHUNK, 1), lambda i: (i, 0)),
        ],
        out_shape=[
            jax.ShapeDtypeStruct((NSC, NP, 64), jnp.float32),
            jax.ShapeDtypeStruct((NP, 1), jnp.float32),
            jax.ShapeDtypeStruct((NP, 1), jnp.float32),
        ],
    )(pdeg_t, xp)


def _tc_mid(t2, dis, W1, b1r, W2p):
    def body(t_ref, dis_ref, w1_ref, b1_ref, w2_ref, g_ref):
        t = jnp.concatenate([t_ref[0], t_ref[1]], axis=1)
        z = t * dis_ref[...]
        h = jnp.dot(z, w1_ref[...], preferred_element_type=jnp.float32)
        h = jnp.maximum(h + b1_ref[...], 0.0)
        g = jnp.dot(h, w2_ref[...], preferred_element_type=jnp.float32)
        g = g * dis_ref[...]
        g_ref[0] = g[:, :32]
        g_ref[1] = g[:, 32:]

    return pl.pallas_call(
        body,
        grid=(NP // CHUNK,),
        in_specs=[
            pl.BlockSpec((NSC, CHUNK, 64), lambda i: (0, i, 0)),
            pl.BlockSpec((CHUNK, 1), lambda i: (i, 0)),
            pl.BlockSpec((F_IN, F_IN), lambda i: (0, 0)),
            pl.BlockSpec((1, F_IN), lambda i: (0, 0)),
            pl.BlockSpec((F_IN, 64), lambda i: (0, 0)),
        ],
        out_specs=pl.BlockSpec((NSC, CHUNK, 32), lambda i: (0, i, 0)),
        out_shape=jax.ShapeDtypeStruct((NSC, NP, 32), jnp.float32),
    )(t2, dis, W1, b1r, W2p)


def _tc_final(t4, dis, b2p):
    def body(t_ref, dis_ref, b2_ref, o_ref):
        t = jnp.concatenate([t_ref[0], t_ref[1]], axis=1)
        z = t * dis_ref[...] + b2_ref[...]
        colid = lax.broadcasted_iota(jnp.int32, z.shape, 1)
        mask = colid < 40
        zm = jnp.where(mask, z, -1e30)
        m = jnp.max(zm, axis=1, keepdims=True)
        e = jnp.where(mask, jnp.exp(z - m), 0.0)
        ssum = jnp.sum(e, axis=1, keepdims=True)
        ls = z - m - jnp.log(ssum)
        o_ref[...] = ls[:, :40]

    return pl.pallas_call(
        body,
        grid=(NP // CHUNK,),
        in_specs=[
            pl.BlockSpec((NSC, CHUNK, 32), lambda i: (0, i, 0)),
            pl.BlockSpec((CHUNK, 1), lambda i: (i, 0)),
            pl.BlockSpec((1, 64), lambda i: (0, 0)),
        ],
        out_specs=pl.BlockSpec((CHUNK, 40), lambda i: (i, 0)),
        out_shape=jax.ShapeDtypeStruct((NP, 40), jnp.float32),
    )(t4, dis, b2p)


def kernel(x, edge_index, W1, b1, W2, b2):
    rows = edge_index[0].astype(jnp.int32)
    cols = edge_index[1].astype(jnp.int32)
    xp = jnp.zeros((NP, F_IN), jnp.float32).at[:N].set(x)
    b1r = b1.reshape(1, F_IN)
    W2p = jnp.zeros((F_IN, 64), jnp.float32).at[:, :40].set(W2)
    b2p = jnp.zeros((1, 64), jnp.float32).at[0, :40].set(b2)

    pdeg = _deg_call(cols)
    pdeg_t = pdeg.reshape(NSC, NP).T
    x0, dis, inv = _tc_pre(pdeg_t, xp)

    m1 = _prop(x0, rows, cols, inv, 64)
    t2 = _prop(m1, rows, cols, None, 64)
    g0 = _tc_mid(t2, dis, W1, b1r, W2p)
    m3 = _prop(g0, rows, cols, inv, 32)
    t4 = _prop(m3, rows, cols, None, 32)
    out = _tc_final(t4, dis, b2p)
    return out[:N]


# speedup vs baseline: 19.8536x; 1.6479x over previous
import jax
import jax.numpy as jnp
from jax import lax
from jax.experimental import pallas as pl
from jax.experimental.pallas import tpu as pltpu
from jax.experimental.pallas import tpu_sc as plsc

N = 10000
F_IN = 128
E = 320000
NSC = 2
NTILE = 16
CHUNK = 640
NP = NTILE * CHUNK
WBC = 160
BE = 80

_MESH = dict(core_axis_name="c", subcore_axis_name="s")
_SC_PARAMS = pltpu.CompilerParams(use_tc_tiling_on_sc=False)


def _sc_mesh():
    return plsc.VectorSubcoreMesh(**_MESH)


def _deg_call(cols):
    ept = E // (NSC * NTILE)
    nb = ept // BE

    def body(cols_h, pdeg_out, dacc, cols_all, colsr, onesr, ibuf, sem):
        c = lax.axis_index("c")
        sid = lax.axis_index("s")
        v0 = sid * CHUNK
        cf = (1 - c).astype(jnp.float32)
        for j in range(BE // 16):
            onesr[pl.ds(j * 16, 16)] = jnp.ones((16,), jnp.float32)
        for j in range(CHUNK // 16):
            ibuf[pl.ds(j * 16, 16)] = jnp.ones((16,), jnp.float32) * cf
        pltpu.sync_copy(ibuf, dacc.at[pl.ds(v0, CHUNK)])
        e0 = c * (E // 2) + sid * ept
        pltpu.sync_copy(cols_h.at[pl.ds(e0, ept)], cols_all)
        plsc.subcore_barrier()

        def blk(b, carry):
            base = b * BE
            for j in range(BE // 16):
                colsr[pl.ds(j * 16, 16)] = cols_all[pl.ds(base + j * 16, 16)]
            pltpu.sync_copy(onesr, dacc.at[colsr], add=True)
            return carry

        lax.fori_loop(0, nb, blk, 0)
        plsc.subcore_barrier()
        pltpu.sync_copy(dacc.at[pl.ds(v0, CHUNK)], ibuf)
        pltpu.sync_copy(ibuf, pdeg_out.at[pl.ds(c * NP + v0, CHUNK)])

    kfn = pl.kernel(
        body,
        out_type=jax.ShapeDtypeStruct((NSC * NP,), jnp.float32),
        mesh=_sc_mesh(),
        compiler_params=_SC_PARAMS,
        scratch_types=[
            pltpu.VMEM_SHARED((NP,), jnp.float32),
            pltpu.VMEM((ept,), jnp.int32),
            pltpu.VMEM((BE,), jnp.int32),
            pltpu.VMEM((BE,), jnp.float32),
            pltpu.VMEM((CHUNK,), jnp.float32),
            pltpu.SemaphoreType.DMA,
        ],
    )
    return kfn(cols)


def _make_prop(w):
    ept = E // NTILE
    nb = ept // BE

    def body(hflat, rows_h, cols_h, s_h, outf, acc, rows_all, cols_all,
             colsr0, colsr1, gidxr0, gidxr1, gbuf0, gbuf1, iobuf, sr,
             semg0, semg1):
        c = lax.axis_index("c")
        sid = lax.axis_index("s")
        v0 = sid * CHUNK
        pltpu.sync_copy(hflat.at[pl.ds(c * NP + v0, CHUNK)],
                        acc.at[pl.ds(v0, CHUNK)])
        pltpu.sync_copy(s_h.at[pl.ds(v0, CHUNK)], sr)
        e0 = sid * ept
        pltpu.sync_copy(rows_h.at[pl.ds(e0, ept)], rows_all)
        pltpu.sync_copy(cols_h.at[pl.ds(e0, ept)], cols_all)
        plsc.subcore_barrier()

        sets = ((colsr0, gidxr0, gbuf0, semg0), (colsr1, gidxr1, gbuf1, semg1))

        def build_fire(b, colsr, gidxr, gbuf, semg):
            base = b * BE
            for j in range(BE // 16):
                rv = rows_all[pl.ds(base + j * 16, 16)]
                gidxr[pl.ds(j * 16, 16)] = rv + c * NP
                colsr[pl.ds(j * 16, 16)] = cols_all[pl.ds(base + j * 16, 16)]
            pltpu.async_copy(hflat.at[gidxr], gbuf, semg)

        build_fire(0, *sets[0])

        def pair(i, carry):
            for par in (0, 1):
                b = 2 * i + par
                colsr, gidxr, gbuf, semg = sets[par]
                ncolsr, ngidxr, ngbuf, nsemg = sets[1 - par]

                @pl.when(b + 1 < nb)
                def _():
                    build_fire(b + 1, ncolsr, ngidxr, ngbuf, nsemg)

                pltpu.make_async_copy(hflat.at[gidxr], gbuf, semg).wait()
                pltpu.sync_copy(gbuf, acc.at[colsr], add=True)
            return carry

        lax.fori_loop(0, nb // 2, pair, 0)
        plsc.subcore_barrier()
        lane = lax.iota(jnp.int32, 16)

        def scale_row(r, s_l):
            for j in range(w // 16):
                iobuf[r, pl.ds(j * 16, 16)] = iobuf[r, pl.ds(j * 16, 16)] * s_l
            if w % 16 == 8:
                tail = iobuf[r, pl.ds(w - 16, 16)]
                iobuf[r, pl.ds(w - 16, 16)] = tail * jnp.where(
                    lane >= 8, s_l, jnp.float32(1.0))

        def wb(q, carry):
            r0 = q * WBC
            pltpu.sync_copy(acc.at[pl.ds(v0 + r0, WBC)], iobuf)

            def grpf(g, carry2):
                sv = sr[pl.ds(r0 + g * 16, 16)]
                for l in range(16):
                    scale_row(g * 16 + l, sv[l])
                return carry2
            lax.fori_loop(0, WBC // 16, grpf, 0)
            pltpu.sync_copy(iobuf, outf.at[pl.ds(c * NP + v0 + r0, WBC)])
            return carry

        lax.fori_loop(0, CHUNK // WBC, wb, 0)

    kfn = pl.kernel(
        body,
        out_type=jax.ShapeDtypeStruct((NSC * NP, w), jnp.float32),
        mesh=_sc_mesh(),
        compiler_params=_SC_PARAMS,
        scratch_types=[
            pltpu.VMEM_SHARED((NP, w), jnp.float32),
            pltpu.VMEM((ept,), jnp.int32),
            pltpu.VMEM((ept,), jnp.int32),
            pltpu.VMEM((BE,), jnp.int32),
            pltpu.VMEM((BE,), jnp.int32),
            pltpu.VMEM((BE,), jnp.int32),
            pltpu.VMEM((BE,), jnp.int32),
            pltpu.VMEM((BE, w), jnp.float32),
            pltpu.VMEM((BE, w), jnp.float32),
            pltpu.VMEM((WBC, w), jnp.float32),
            pltpu.VMEM((CHUNK,), jnp.float32),
            pltpu.SemaphoreType.DMA,
            pltpu.SemaphoreType.DMA,
        ],
    )
    return kfn


def _prop(h3, rows, cols, s, w):
    hflat = h3.reshape(NSC * NP, w)
    out = _make_prop(w)(hflat, rows, cols, s.reshape(NP))
    return out.reshape(NSC, NP, w)


def _tc_pre(pdeg_t, xp):
    def body(pd_ref, x_ref, x0_ref, dis_ref, inv_ref):
        pd = pd_ref[...]
        deg = pd[:, 0:1] + pd[:, 1:2] - 1.0
        dis = lax.rsqrt(deg)
        dis_ref[...] = dis
        inv_ref[...] = 1.0 / deg
        x0 = x_ref[...] * dis
        x0_ref[0] = x0[:, :64]
        x0_ref[1] = x0[:, 64:]

    return pl.pallas_call(
        body,
        grid=(NP // CHUNK,),
        in_specs=[
            pl.BlockSpec((CHUNK, NSC), lambda i: (i, 0)),
            pl.BlockSpec((CHUNK, F_IN), lambda i: (i, 0)),
        ],
        out_specs=[
            pl.BlockSpec((NSC, CHUNK, 64), lambda i: (0, i, 0)),
            pl.BlockSpec((CHUNK, 1), lambda i: (i, 0)),
            pl.BlockSpec((CHUNK, 1), lambda i: (i, 0)),
        ],
        out_shape=[
            jax.ShapeDtypeStruct((NSC, NP, 64), jnp.float32),
            jax.ShapeDtypeStruct((NP, 1), jnp.float32),
            jax.ShapeDtypeStruct((NP, 1), jnp.float32),
        ],
    )(pdeg_t, xp)


def _tc_mid(t2, dis, W1, b1r, W2p):
    def body(t_ref, dis_ref, w1_ref, b1_ref, w2_ref, g_ref):
        z = jnp.concatenate([t_ref[0], t_ref[1]], axis=1)
        h = jnp.dot(z, w1_ref[...], preferred_element_type=jnp.float32)
        h = jnp.maximum(h + b1_ref[...], 0.0)
        g = jnp.dot(h, w2_ref[...], preferred_element_type=jnp.float32)
        g = g * dis_ref[...]
        g_ref[0] = g[:, :24]
        g_ref[1] = g[:, 24:]

    return pl.pallas_call(
        body,
        grid=(NP // CHUNK,),
        in_specs=[
            pl.BlockSpec((NSC, CHUNK, 64), lambda i: (0, i, 0)),
            pl.BlockSpec((CHUNK, 1), lambda i: (i, 0)),
            pl.BlockSpec((F_IN, F_IN), lambda i: (0, 0)),
            pl.BlockSpec((1, F_IN), lambda i: (0, 0)),
            pl.BlockSpec((F_IN, 48), lambda i: (0, 0)),
        ],
        out_specs=pl.BlockSpec((NSC, CHUNK, 24), lambda i: (0, i, 0)),
        out_shape=jax.ShapeDtypeStruct((NSC, NP, 24), jnp.float32),
    )(t2, dis, W1, b1r, W2p)


def _tc_final(t4, b2p):
    def body(t_ref, b2_ref, o_ref):
        t = jnp.concatenate([t_ref[0], t_ref[1]], axis=1)
        z = t + b2_ref[...]
        colid = lax.broadcasted_iota(jnp.int32, z.shape, 1)
        mask = colid < 40
        zm = jnp.where(mask, z, -1e30)
        m = jnp.max(zm, axis=1, keepdims=True)
        e = jnp.where(mask, jnp.exp(z - m), 0.0)
        ssum = jnp.sum(e, axis=1, keepdims=True)
        ls = z - m - jnp.log(ssum)
        o_ref[...] = ls[:, :40]

    return pl.pallas_call(
        body,
        grid=(NP // CHUNK,),
        in_specs=[
            pl.BlockSpec((NSC, CHUNK, 24), lambda i: (0, i, 0)),
            pl.BlockSpec((1, 48), lambda i: (0, 0)),
        ],
        out_specs=pl.BlockSpec((CHUNK, 40), lambda i: (i, 0)),
        out_shape=jax.ShapeDtypeStruct((NP, 40), jnp.float32),
    )(t4, b2p)


def kernel(x, edge_index, W1, b1, W2, b2):
    rows = edge_index[0].astype(jnp.int32)
    cols = edge_index[1].astype(jnp.int32)
    xp = jnp.zeros((NP, F_IN), jnp.float32).at[:N].set(x)
    b1r = b1.reshape(1, F_IN)
    W2p = jnp.zeros((F_IN, 48), jnp.float32).at[:, :40].set(W2)
    b2p = jnp.zeros((1, 48), jnp.float32).at[0, :40].set(b2)

    pdeg = _deg_call(cols)
    pdeg_t = pdeg.reshape(NSC, NP).T
    x0, dis, inv = _tc_pre(pdeg_t, xp)

    m1 = _prop(x0, rows, cols, inv, 64)
    z = _prop(m1, rows, cols, dis, 64)
    g0 = _tc_mid(z, dis, W1, b1r, W2p)
    m3 = _prop(g0, rows, cols, inv, 24)
    z2 = _prop(m3, rows, cols, dis, 24)
    out = _tc_final(z2, b2p)
    return out[:N]
